# baseline (speedup 1.0000x reference)
import functools

import jax
import jax.numpy as jnp
from jax import lax
from jax.experimental import pallas as pl
from jax.experimental.pallas import tpu as pltpu
from jax.experimental.pallas import tpu_sc as plsc

N = 10000
E = 320000
D = 128
H = 128
P = 8192

NC = 2
NS = 16
L = 16
NW = NC * NS
EPW = E // NW
C = 128
NCHUNK = EPW // C
TAIL = EPW - NCHUNK * C
RPS = 624
RTAIL = N - NS * RPS
DW = 16
QS = (4 * P) // NW
SCH = QS // C


def _sc_agg_body(table, src, dst, zagg, zdeg1,
                 aggp, degp,
                 sidx0, sidx1, didx0, didx1,
                 rows0, rows1,
                 tsidx, tdidx, trows,
                 deg_v, agg_s, gsem0, gsem1, isem0, isem1):
    sidx = (sidx0, sidx1)
    didx = (didx0, didx1)
    rows = (rows0, rows1)
    gsem = (gsem0, gsem1)
    isem = (isem0, isem1)
    c = lax.axis_index("c")
    s = lax.axis_index("s")
    wid = s * NC + c
    r0 = s * RPS
    pltpu.sync_copy(zagg.at[pl.ds(r0, RPS)], agg_s.at[pl.ds(r0, RPS)])
    pltpu.sync_copy(zdeg1, deg_v)

    @pl.when(s == NS - 1)
    def _():
        rt = NS * RPS
        pltpu.sync_copy(zagg.at[pl.ds(rt, RTAIL)], agg_s.at[pl.ds(rt, RTAIL)])

    plsc.subcore_barrier()

    base = wid * EPW
    ones16 = jnp.ones((L,), jnp.float32)

    def load_idx_async(k, b):
        off = base + k * C
        pltpu.async_copy(src.at[pl.ds(off, C)], sidx[b], isem[b])
        pltpu.async_copy(dst.at[pl.ds(off, C)], didx[b], isem[b])

    def wait_idx(b):
        pltpu.make_async_copy(src.at[pl.ds(0, C)], sidx[b], isem[b]).wait()
        pltpu.make_async_copy(dst.at[pl.ds(0, C)], didx[b], isem[b]).wait()

    load_idx_async(0, 0)
    load_idx_async(1, 1)
    wait_idx(0)
    pltpu.async_copy(table.at[sidx[0]], rows[0], gsem[0])

    def step(k, b):
        bo = 1 - b
        pltpu.make_async_copy(table.at[sidx[b]], rows[b], gsem[b]).wait()
        pltpu.sync_copy(rows[b], agg_s.at[didx[b]], add=True)
        for j in range(C // L):
            plsc.addupdate_scatter(deg_v, [didx[b][pl.ds(j * L, L)]], ones16)

        @pl.when(k + 2 < NCHUNK)
        def _():
            load_idx_async(k + 2, b)

        @pl.when(k + 1 < NCHUNK)
        def _():
            wait_idx(bo)
            pltpu.async_copy(table.at[sidx[bo]], rows[bo], gsem[bo])

    def outer(i, carry):
        step(2 * i, 0)
        step(2 * i + 1, 1)
        return carry

    lax.fori_loop(0, NCHUNK // 2, outer, 0)

    offt = base + NCHUNK * C
    pltpu.sync_copy(src.at[pl.ds(offt, TAIL)], tsidx)
    pltpu.sync_copy(dst.at[pl.ds(offt, TAIL)], tdidx)
    pltpu.async_copy(table.at[tsidx], trows, gsem[0]).wait()
    pltpu.sync_copy(trows, agg_s.at[tdidx], add=True)
    plsc.addupdate_scatter(deg_v, [tdidx[...]], ones16)

    pltpu.sync_copy(deg_v, degp.at[pl.ds(wid * N, N)])

    plsc.subcore_barrier()
    pltpu.sync_copy(agg_s.at[pl.ds(r0, RPS)], aggp.at[c, pl.ds(r0, RPS)])

    @pl.when(s == NS - 1)
    def _():
        rt = NS * RPS
        pltpu.sync_copy(agg_s.at[pl.ds(rt, RTAIL)], aggp.at[c, pl.ds(rt, RTAIL)])


_sc_agg = pl.kernel(
    _sc_agg_body,
    out_type=(jax.ShapeDtypeStruct((NC, N, H), jnp.float32),
              jax.ShapeDtypeStruct((NW * N,), jnp.float32)),
    mesh=plsc.VectorSubcoreMesh(core_axis_name="c", subcore_axis_name="s"),
    scratch_types=[
        pltpu.VMEM((C,), jnp.int32),
        pltpu.VMEM((C,), jnp.int32),
        pltpu.VMEM((C,), jnp.int32),
        pltpu.VMEM((C,), jnp.int32),
        pltpu.VMEM((C, H), jnp.float32),
        pltpu.VMEM((C, H), jnp.float32),
        pltpu.VMEM((TAIL,), jnp.int32),
        pltpu.VMEM((TAIL,), jnp.int32),
        pltpu.VMEM((TAIL, H), jnp.float32),
        pltpu.VMEM((N,), jnp.float32),
        pltpu.VMEM_SHARED((N, H), jnp.float32),
        pltpu.SemaphoreType.DMA,
        pltpu.SemaphoreType.DMA,
        pltpu.SemaphoreType.DMA,
        pltpu.SemaphoreType.DMA,
    ],
    compiler_params=pltpu.CompilerParams(needs_layout_passes=False),
)


def _sc_score_body(hitem, idx_all, bias,
                   rows_out, bvals_out,
                   idx_v, rows_v, bias_v, bch, gsem):
    c = lax.axis_index("c")
    s = lax.axis_index("s")
    wid = s * NC + c
    base = wid * QS
    pltpu.sync_copy(bias, bias_v)

    def chunk(k, carry):
        off = base + k * C
        pltpu.sync_copy(idx_all.at[pl.ds(off, C)], idx_v)
        pltpu.async_copy(hitem.at[idx_v], rows_v, gsem).wait()
        pltpu.sync_copy(rows_v, rows_out.at[pl.ds(off, C)])
        for j in range(C // L):
            bidx = idx_v[pl.ds(j * L, L)]
            bch[pl.ds(j * L, L)] = plsc.load_gather(bias_v, [bidx])
        pltpu.sync_copy(bch, bvals_out.at[pl.ds(off, C)])
        return carry

    lax.fori_loop(0, SCH, chunk, 0)


_sc_score = pl.kernel(
    _sc_score_body,
    out_type=(jax.ShapeDtypeStruct((4 * P, H), jnp.float32),
              jax.ShapeDtypeStruct((4 * P,), jnp.float32)),
    mesh=plsc.VectorSubcoreMesh(core_axis_name="c", subcore_axis_name="s"),
    scratch_types=[
        pltpu.VMEM((C,), jnp.int32),
        pltpu.VMEM((C, H), jnp.float32),
        pltpu.VMEM((N,), jnp.float32),
        pltpu.VMEM((C,), jnp.float32),
        pltpu.SemaphoreType.DMA,
    ],
    compiler_params=pltpu.CompilerParams(needs_layout_passes=False),
)


R = 2000
GRID = N // R


def _mm(a, b):
    return jnp.dot(a, b, preferred_element_type=jnp.float32)


def _tc_prep_body(x_ref, wp_ref, emb_ref, q1_ref, bq1_ref, h0_ref, n1_ref):
    h0 = _mm(x_ref[...], wp_ref[...]) + emb_ref[...]
    h0_ref[...] = h0
    n1_ref[...] = jnp.maximum(_mm(h0, q1_ref[...]) + bq1_ref[...], 0.0)


_tc_prep = pl.pallas_call(
    _tc_prep_body,
    grid=(GRID,),
    in_specs=[
        pl.BlockSpec((R, D), lambda i: (i, 0)),
        pl.BlockSpec((D, H), lambda i: (0, 0)),
        pl.BlockSpec((R, H), lambda i: (i, 0)),
        pl.BlockSpec((H, H), lambda i: (0, 0)),
        pl.BlockSpec((1, H), lambda i: (0, 0)),
    ],
    out_specs=[
        pl.BlockSpec((R, H), lambda i: (i, 0)),
        pl.BlockSpec((R, H), lambda i: (i, 0)),
    ],
    out_shape=[
        jax.ShapeDtypeStruct((N, H), jnp.float32),
        jax.ShapeDtypeStruct((N, H), jnp.float32),
    ],
)


def _tc_deg_body(degp_ref, deg_ref):
    deg_col = lax.dot_general(
        degp_ref[...], jnp.ones((NW, 1), jnp.float32),
        dimension_numbers=(((0,), (0,)), ((), ())),
        preferred_element_type=jnp.float32)
    deg_ref[...] = jnp.maximum(deg_col, 1.0)


_tc_deg = pl.pallas_call(
    _tc_deg_body,
    out_shape=jax.ShapeDtypeStruct((N, 1), jnp.float32),
)


def _combine(aggp_ref, deg_ref, h_ref, w_ref, bw_ref):
    agg = aggp_ref[0] + aggp_ref[1]
    aggm = agg / deg_ref[...]
    z = jnp.maximum(
        _mm(aggm, w_ref[0:H]) + _mm(h_ref[...], w_ref[H:2 * H]) + bw_ref[...],
        0.0)
    zn = jnp.maximum(jnp.sqrt(jnp.sum(z * z, axis=1, keepdims=True)), 1e-6)
    return z / zn


def _tc_comb1_body(aggp_ref, degp_ref, h0_ref, w1_ref, bw1_ref, q2_ref,
                   bq2_ref, h1_ref, n2_ref):
    h1 = _combine(aggp_ref, degp_ref, h0_ref, w1_ref, bw1_ref)
    h1_ref[...] = h1
    n2_ref[...] = jnp.maximum(_mm(h1, q2_ref[...]) + bq2_ref[...], 0.0)


_tc_comb1 = pl.pallas_call(
    _tc_comb1_body,
    grid=(GRID,),
    in_specs=[
        pl.BlockSpec((NC, R, H), lambda i: (0, i, 0)),
        pl.BlockSpec((R, 1), lambda i: (i, 0)),
        pl.BlockSpec((R, H), lambda i: (i, 0)),
        pl.BlockSpec((2 * H, H), lambda i: (0, 0)),
        pl.BlockSpec((1, H), lambda i: (0, 0)),
        pl.BlockSpec((H, H), lambda i: (0, 0)),
        pl.BlockSpec((1, H), lambda i: (0, 0)),
    ],
    out_specs=[
        pl.BlockSpec((R, H), lambda i: (i, 0)),
        pl.BlockSpec((R, H), lambda i: (i, 0)),
    ],
    out_shape=[
        jax.ShapeDtypeStruct((N, H), jnp.float32),
        jax.ShapeDtypeStruct((N, H), jnp.float32),
    ],
)


def _tc_comb2_body(aggp_ref, degp_ref, h1_ref, h0_ref, w2_ref, bw2_ref,
                   hitem_ref):
    h2 = _combine(aggp_ref, degp_ref, h1_ref, w2_ref, bw2_ref)
    hitem_ref[...] = h0_ref[...] + h2


_tc_comb2 = pl.pallas_call(
    _tc_comb2_body,
    grid=(GRID,),
    in_specs=[
        pl.BlockSpec((NC, R, H), lambda i: (0, i, 0)),
        pl.BlockSpec((R, 1), lambda i: (i, 0)),
        pl.BlockSpec((R, H), lambda i: (i, 0)),
        pl.BlockSpec((R, H), lambda i: (i, 0)),
        pl.BlockSpec((2 * H, H), lambda i: (0, 0)),
        pl.BlockSpec((1, H), lambda i: (0, 0)),
    ],
    out_specs=pl.BlockSpec((R, H), lambda i: (i, 0)),
    out_shape=jax.ShapeDtypeStruct((N, H), jnp.float32),
)


RP = 2048
SGRID = P // RP


def _tc_score_body(rows_ref, bv_ref, out_ref):
    pos = jnp.sum(rows_ref[0] * rows_ref[1], axis=1) + bv_ref[0] + bv_ref[1]
    neg = jnp.sum(rows_ref[2] * rows_ref[3], axis=1) + bv_ref[2] + bv_ref[3]
    out_ref[...] = jnp.maximum(neg - pos + 1.0, 0.0)[None, :]


_tc_score = pl.pallas_call(
    _tc_score_body,
    grid=(SGRID,),
    in_specs=[
        pl.BlockSpec((4, RP, H), lambda i: (0, i, 0)),
        pl.BlockSpec((4, RP), lambda i: (0, i)),
    ],
    out_specs=pl.BlockSpec((1, RP), lambda i: (0, i)),
    out_shape=jax.ShapeDtypeStruct((1, P), jnp.float32),
)


def kernel(x, edge_index, pos_edges, neg_edges, item_emb, W_proj, Q1, bq1,
           W1, bw1, Q2, bq2, W2, bw2, bias):
    src = edge_index[0]
    dst = edge_index[1]
    zagg = jnp.zeros((N, H), jnp.float32)
    zdeg1 = jnp.zeros((N,), jnp.float32)

    h0, n1 = _tc_prep(x, W_proj, item_emb, Q1, bq1.reshape(1, H))
    aggp1, degf = _sc_agg(n1, src, dst, zagg, zdeg1)
    degp = _tc_deg(degf.reshape(NW, N))
    h1, n2 = _tc_comb1(aggp1, degp, h0, W1, bw1.reshape(1, H), Q2,
                       bq2.reshape(1, H))
    aggp2, _ = _sc_agg(n2, src, dst, zagg, zdeg1)
    h_item = _tc_comb2(aggp2, degp, h1, h0, W2, bw2.reshape(1, H))

    idx_all = jnp.concatenate(
        [pos_edges[0], pos_edges[1], neg_edges[0], neg_edges[1]])
    rows, bvals = _sc_score(h_item, idx_all, bias)
    out = _tc_score(rows.reshape(4, P, H), bvals.reshape(4, P))
    return out.reshape(P)


# speedup vs baseline: 1.4104x; 1.4104x over previous
import functools

import jax
import jax.numpy as jnp
from jax import lax
from jax.experimental import pallas as pl
from jax.experimental.pallas import tpu as pltpu
from jax.experimental.pallas import tpu_sc as plsc

N = 10000
E = 320000
D = 128
H = 128
P = 8192

NC = 2
NS = 16
L = 16
NW = NC * NS
EPW = E // NW
C = 128
NCHUNK = EPW // C
TAIL = EPW - NCHUNK * C
RPS = 624
RTAIL = N - NS * RPS
DW = 16
QS = (4 * P) // NW
SCH = QS // C


def _sc_agg_body(table, src, dst, zagg, zdeg1,
                 aggp, degp,
                 sidx0, sidx1, sidx2, didx0, didx1, didx2,
                 rows0, rows1,
                 tsidx, tdidx, trows,
                 deg_v, agg_s, gsem0, gsem1, isem0, isem1, isem2):
    sidx = (sidx0, sidx1, sidx2)
    didx = (didx0, didx1, didx2)
    rows = (rows0, rows1)
    gsem = (gsem0, gsem1)
    isem = (isem0, isem1, isem2)
    c = lax.axis_index("c")
    s = lax.axis_index("s")
    wid = s * NC + c
    r0 = s * RPS
    pltpu.sync_copy(zagg.at[pl.ds(r0, RPS)], agg_s.at[pl.ds(r0, RPS)])
    pltpu.sync_copy(zdeg1, deg_v)

    @pl.when(s == NS - 1)
    def _():
        rt = NS * RPS
        pltpu.sync_copy(zagg.at[pl.ds(rt, RTAIL)], agg_s.at[pl.ds(rt, RTAIL)])

    plsc.subcore_barrier()

    base = wid * EPW
    ones16 = jnp.ones((L,), jnp.float32)

    def load_idx_async(k, bi):
        off = base + k * C
        pltpu.async_copy(src.at[pl.ds(off, C)], sidx[bi], isem[bi])
        pltpu.async_copy(dst.at[pl.ds(off, C)], didx[bi], isem[bi])

    def wait_idx(bi):
        pltpu.make_async_copy(src.at[pl.ds(0, C)], sidx[bi], isem[bi]).wait()
        pltpu.make_async_copy(dst.at[pl.ds(0, C)], didx[bi], isem[bi]).wait()

    load_idx_async(0, 0)
    load_idx_async(1, 1)
    load_idx_async(2, 2)
    wait_idx(0)
    pltpu.async_copy(table.at[sidx[0]], rows[0], gsem[0])
    wait_idx(1)
    pltpu.async_copy(table.at[sidx[1]], rows[1], gsem[1])

    def step(k, br, bi):
        pltpu.make_async_copy(table.at[sidx[bi]], rows[br], gsem[br]).wait()
        pltpu.sync_copy(rows[br], agg_s.at[didx[bi]], add=True)
        for j in range(C // L):
            plsc.addupdate_scatter(deg_v, [didx[bi][pl.ds(j * L, L)]], ones16)

        @pl.when(k + 3 < NCHUNK)
        def _():
            load_idx_async(k + 3, bi)

        @pl.when(k + 2 < NCHUNK)
        def _():
            bi2 = (bi + 2) % 3
            wait_idx(bi2)
            pltpu.async_copy(table.at[sidx[bi2]], rows[br], gsem[br])

    def outer(i, carry):
        for u in range(6):
            step(6 * i + u, u % 2, u % 3)
        return carry

    lax.fori_loop(0, NCHUNK // 6, outer, 0)

    offt = base + NCHUNK * C
    pltpu.sync_copy(src.at[pl.ds(offt, TAIL)], tsidx)
    pltpu.sync_copy(dst.at[pl.ds(offt, TAIL)], tdidx)
    pltpu.async_copy(table.at[tsidx], trows, gsem[0]).wait()
    pltpu.sync_copy(trows, agg_s.at[tdidx], add=True)
    plsc.addupdate_scatter(deg_v, [tdidx[...]], ones16)

    pltpu.sync_copy(deg_v, degp.at[pl.ds(wid * N, N)])

    plsc.subcore_barrier()
    pltpu.sync_copy(agg_s.at[pl.ds(r0, RPS)], aggp.at[c, pl.ds(r0, RPS)])

    @pl.when(s == NS - 1)
    def _():
        rt = NS * RPS
        pltpu.sync_copy(agg_s.at[pl.ds(rt, RTAIL)], aggp.at[c, pl.ds(rt, RTAIL)])


_sc_agg = pl.kernel(
    _sc_agg_body,
    out_type=(jax.ShapeDtypeStruct((NC, N, H), jnp.float32),
              jax.ShapeDtypeStruct((NW * N,), jnp.float32)),
    mesh=plsc.VectorSubcoreMesh(core_axis_name="c", subcore_axis_name="s"),
    scratch_types=[
        pltpu.VMEM((C,), jnp.int32),
        pltpu.VMEM((C,), jnp.int32),
        pltpu.VMEM((C,), jnp.int32),
        pltpu.VMEM((C,), jnp.int32),
        pltpu.VMEM((C,), jnp.int32),
        pltpu.VMEM((C,), jnp.int32),
        pltpu.VMEM((C, H), jnp.float32),
        pltpu.VMEM((C, H), jnp.float32),
        pltpu.VMEM((TAIL,), jnp.int32),
        pltpu.VMEM((TAIL,), jnp.int32),
        pltpu.VMEM((TAIL, H), jnp.float32),
        pltpu.VMEM((N,), jnp.float32),
        pltpu.VMEM_SHARED((N, H), jnp.float32),
        pltpu.SemaphoreType.DMA,
        pltpu.SemaphoreType.DMA,
        pltpu.SemaphoreType.DMA,
        pltpu.SemaphoreType.DMA,
        pltpu.SemaphoreType.DMA,
    ],
    compiler_params=pltpu.CompilerParams(needs_layout_passes=False),
)


def _sc_score_body(hitem, idx_all, bias,
                   rows_out, bvals_out,
                   idx_v, rows_v, bias_v, bch, gsem):
    c = lax.axis_index("c")
    s = lax.axis_index("s")
    wid = s * NC + c
    base = wid * QS
    pltpu.sync_copy(bias, bias_v)

    def chunk(k, carry):
        off = base + k * C
        pltpu.sync_copy(idx_all.at[pl.ds(off, C)], idx_v)
        pltpu.async_copy(hitem.at[idx_v], rows_v, gsem).wait()
        pltpu.sync_copy(rows_v, rows_out.at[pl.ds(off, C)])
        for j in range(C // L):
            bidx = idx_v[pl.ds(j * L, L)]
            bch[pl.ds(j * L, L)] = plsc.load_gather(bias_v, [bidx])
        pltpu.sync_copy(bch, bvals_out.at[pl.ds(off, C)])
        return carry

    lax.fori_loop(0, SCH, chunk, 0)


_sc_score = pl.kernel(
    _sc_score_body,
    out_type=(jax.ShapeDtypeStruct((4 * P, H), jnp.float32),
              jax.ShapeDtypeStruct((4 * P,), jnp.float32)),
    mesh=plsc.VectorSubcoreMesh(core_axis_name="c", subcore_axis_name="s"),
    scratch_types=[
        pltpu.VMEM((C,), jnp.int32),
        pltpu.VMEM((C, H), jnp.float32),
        pltpu.VMEM((N,), jnp.float32),
        pltpu.VMEM((C,), jnp.float32),
        pltpu.SemaphoreType.DMA,
    ],
    compiler_params=pltpu.CompilerParams(needs_layout_passes=False),
)


R = 2000
GRID = N // R


def _mm(a, b):
    return jnp.dot(a, b, preferred_element_type=jnp.float32)


def _tc_prep_body(x_ref, wp_ref, emb_ref, q1_ref, bq1_ref, h0_ref, n1_ref):
    h0 = _mm(x_ref[...], wp_ref[...]) + emb_ref[...]
    h0_ref[...] = h0
    n1_ref[...] = jnp.maximum(_mm(h0, q1_ref[...]) + bq1_ref[...], 0.0)


_tc_prep = pl.pallas_call(
    _tc_prep_body,
    grid=(GRID,),
    in_specs=[
        pl.BlockSpec((R, D), lambda i: (i, 0)),
        pl.BlockSpec((D, H), lambda i: (0, 0)),
        pl.BlockSpec((R, H), lambda i: (i, 0)),
        pl.BlockSpec((H, H), lambda i: (0, 0)),
        pl.BlockSpec((1, H), lambda i: (0, 0)),
    ],
    out_specs=[
        pl.BlockSpec((R, H), lambda i: (i, 0)),
        pl.BlockSpec((R, H), lambda i: (i, 0)),
    ],
    out_shape=[
        jax.ShapeDtypeStruct((N, H), jnp.float32),
        jax.ShapeDtypeStruct((N, H), jnp.float32),
    ],
)


def _tc_deg_body(degp_ref, deg_ref):
    deg_col = lax.dot_general(
        degp_ref[...], jnp.ones((NW, 1), jnp.float32),
        dimension_numbers=(((0,), (0,)), ((), ())),
        preferred_element_type=jnp.float32)
    deg_ref[...] = jnp.maximum(deg_col, 1.0)


_tc_deg = pl.pallas_call(
    _tc_deg_body,
    out_shape=jax.ShapeDtypeStruct((N, 1), jnp.float32),
)


def _combine(aggp_ref, deg_ref, h_ref, w_ref, bw_ref):
    agg = aggp_ref[0] + aggp_ref[1]
    aggm = agg / deg_ref[...]
    z = jnp.maximum(
        _mm(aggm, w_ref[0:H]) + _mm(h_ref[...], w_ref[H:2 * H]) + bw_ref[...],
        0.0)
    zn = jnp.maximum(jnp.sqrt(jnp.sum(z * z, axis=1, keepdims=True)), 1e-6)
    return z / zn


def _tc_comb1_body(aggp_ref, degp_ref, h0_ref, w1_ref, bw1_ref, q2_ref,
                   bq2_ref, h1_ref, n2_ref):
    h1 = _combine(aggp_ref, degp_ref, h0_ref, w1_ref, bw1_ref)
    h1_ref[...] = h1
    n2_ref[...] = jnp.maximum(_mm(h1, q2_ref[...]) + bq2_ref[...], 0.0)


_tc_comb1 = pl.pallas_call(
    _tc_comb1_body,
    grid=(GRID,),
    in_specs=[
        pl.BlockSpec((NC, R, H), lambda i: (0, i, 0)),
        pl.BlockSpec((R, 1), lambda i: (i, 0)),
        pl.BlockSpec((R, H), lambda i: (i, 0)),
        pl.BlockSpec((2 * H, H), lambda i: (0, 0)),
        pl.BlockSpec((1, H), lambda i: (0, 0)),
        pl.BlockSpec((H, H), lambda i: (0, 0)),
        pl.BlockSpec((1, H), lambda i: (0, 0)),
    ],
    out_specs=[
        pl.BlockSpec((R, H), lambda i: (i, 0)),
        pl.BlockSpec((R, H), lambda i: (i, 0)),
    ],
    out_shape=[
        jax.ShapeDtypeStruct((N, H), jnp.float32),
        jax.ShapeDtypeStruct((N, H), jnp.float32),
    ],
)


def _tc_comb2_body(aggp_ref, degp_ref, h1_ref, h0_ref, w2_ref, bw2_ref,
                   hitem_ref):
    h2 = _combine(aggp_ref, degp_ref, h1_ref, w2_ref, bw2_ref)
    hitem_ref[...] = h0_ref[...] + h2


_tc_comb2 = pl.pallas_call(
    _tc_comb2_body,
    grid=(GRID,),
    in_specs=[
        pl.BlockSpec((NC, R, H), lambda i: (0, i, 0)),
        pl.BlockSpec((R, 1), lambda i: (i, 0)),
        pl.BlockSpec((R, H), lambda i: (i, 0)),
        pl.BlockSpec((R, H), lambda i: (i, 0)),
        pl.BlockSpec((2 * H, H), lambda i: (0, 0)),
        pl.BlockSpec((1, H), lambda i: (0, 0)),
    ],
    out_specs=pl.BlockSpec((R, H), lambda i: (i, 0)),
    out_shape=jax.ShapeDtypeStruct((N, H), jnp.float32),
)


RP = 2048
SGRID = P // RP


def _tc_score_body(rows_ref, bv_ref, out_ref):
    pos = jnp.sum(rows_ref[0] * rows_ref[1], axis=1) + bv_ref[0] + bv_ref[1]
    neg = jnp.sum(rows_ref[2] * rows_ref[3], axis=1) + bv_ref[2] + bv_ref[3]
    out_ref[...] = jnp.maximum(neg - pos + 1.0, 0.0)[None, :]


_tc_score = pl.pallas_call(
    _tc_score_body,
    grid=(SGRID,),
    in_specs=[
        pl.BlockSpec((4, RP, H), lambda i: (0, i, 0)),
        pl.BlockSpec((4, RP), lambda i: (0, i)),
    ],
    out_specs=pl.BlockSpec((1, RP), lambda i: (0, i)),
    out_shape=jax.ShapeDtypeStruct((1, P), jnp.float32),
)


def kernel(x, edge_index, pos_edges, neg_edges, item_emb, W_proj, Q1, bq1,
           W1, bw1, Q2, bq2, W2, bw2, bias):
    src = edge_index[0]
    dst = edge_index[1]
    zagg = jnp.zeros((N, H), jnp.float32)
    zdeg1 = jnp.zeros((N,), jnp.float32)

    h0, n1 = _tc_prep(x, W_proj, item_emb, Q1, bq1.reshape(1, H))
    aggp1, degf = _sc_agg(n1, src, dst, zagg, zdeg1)
    degp = _tc_deg(degf.reshape(NW, N))
    h1, n2 = _tc_comb1(aggp1, degp, h0, W1, bw1.reshape(1, H), Q2,
                       bq2.reshape(1, H))
    aggp2, _ = _sc_agg(n2, src, dst, zagg, zdeg1)
    h_item = _tc_comb2(aggp2, degp, h1, h0, W2, bw2.reshape(1, H))

    idx_all = jnp.concatenate(
        [pos_edges[0], pos_edges[1], neg_edges[0], neg_edges[1]])
    rows, bvals = _sc_score(h_item, idx_all, bias)
    out = _tc_score(rows.reshape(4, P, H), bvals.reshape(4, P))
    return out.reshape(P)


# speedup vs baseline: 1.4255x; 1.0107x over previous
import functools

import jax
import jax.numpy as jnp
from jax import lax
from jax.experimental import pallas as pl
from jax.experimental.pallas import tpu as pltpu
from jax.experimental.pallas import tpu_sc as plsc

N = 10000
E = 320000
D = 128
H = 128
P = 8192

NC = 2
NS = 16
L = 16
NW = NC * NS
EPW = E // NW
C = 128
NCHUNK = EPW // C
TAIL = EPW - NCHUNK * C
RPS = 624
RTAIL = N - NS * RPS
DW = 16
QS = (4 * P) // NW
SCH = QS // C


def _sc_agg_body(with_deg, *refs):
    if with_deg:
        (table, src, dst, zagg, zdeg1, aggp, degp,
         sidx0, sidx1, sidx2, didx0, didx1, didx2, rows0, rows1,
         tsidx, tdidx, trows, deg_v, agg_s,
         gsem0, gsem1, isem0, isem1, isem2) = refs
    else:
        (table, src, dst, zagg, aggp,
         sidx0, sidx1, sidx2, didx0, didx1, didx2, rows0, rows1,
         tsidx, tdidx, trows, agg_s,
         gsem0, gsem1, isem0, isem1, isem2) = refs
    sidx = (sidx0, sidx1, sidx2)
    didx = (didx0, didx1, didx2)
    rows = (rows0, rows1)
    gsem = (gsem0, gsem1)
    isem = (isem0, isem1, isem2)
    c = lax.axis_index("c")
    s = lax.axis_index("s")
    wid = s * NC + c
    r0 = s * RPS
    pltpu.sync_copy(zagg.at[pl.ds(r0, RPS)], agg_s.at[pl.ds(r0, RPS)])
    if with_deg:
        pltpu.sync_copy(zdeg1, deg_v)

    @pl.when(s == NS - 1)
    def _():
        rt = NS * RPS
        pltpu.sync_copy(zagg.at[pl.ds(rt, RTAIL)], agg_s.at[pl.ds(rt, RTAIL)])

    plsc.subcore_barrier()

    base = wid * EPW
    ones16 = jnp.ones((L,), jnp.float32)

    def load_idx_async(k, bi):
        off = base + k * C
        pltpu.async_copy(src.at[pl.ds(off, C)], sidx[bi], isem[bi])
        pltpu.async_copy(dst.at[pl.ds(off, C)], didx[bi], isem[bi])

    def wait_idx(bi):
        pltpu.make_async_copy(src.at[pl.ds(0, C)], sidx[bi], isem[bi]).wait()
        pltpu.make_async_copy(dst.at[pl.ds(0, C)], didx[bi], isem[bi]).wait()

    load_idx_async(0, 0)
    load_idx_async(1, 1)
    load_idx_async(2, 2)
    wait_idx(0)
    pltpu.async_copy(table.at[sidx[0]], rows[0], gsem[0])
    wait_idx(1)
    pltpu.async_copy(table.at[sidx[1]], rows[1], gsem[1])

    def step(k, br, bi):
        pltpu.make_async_copy(table.at[sidx[bi]], rows[br], gsem[br]).wait()
        pltpu.sync_copy(rows[br], agg_s.at[didx[bi]], add=True)
        if with_deg:
            for j in range(C // L):
                plsc.addupdate_scatter(deg_v, [didx[bi][pl.ds(j * L, L)]],
                                       ones16)

        @pl.when(k + 3 < NCHUNK)
        def _():
            load_idx_async(k + 3, bi)

        @pl.when(k + 2 < NCHUNK)
        def _():
            bi2 = (bi + 2) % 3
            wait_idx(bi2)
            pltpu.async_copy(table.at[sidx[bi2]], rows[br], gsem[br])

    def outer(i, carry):
        for u in range(6):
            step(6 * i + u, u % 2, u % 3)
        return carry

    lax.fori_loop(0, NCHUNK // 6, outer, 0)

    offt = base + NCHUNK * C
    pltpu.sync_copy(src.at[pl.ds(offt, TAIL)], tsidx)
    pltpu.sync_copy(dst.at[pl.ds(offt, TAIL)], tdidx)
    pltpu.async_copy(table.at[tsidx], trows, gsem[0]).wait()
    pltpu.sync_copy(trows, agg_s.at[tdidx], add=True)
    if with_deg:
        plsc.addupdate_scatter(deg_v, [tdidx[...]], ones16)
        pltpu.sync_copy(deg_v, degp.at[pl.ds(wid * N, N)])

    plsc.subcore_barrier()
    pltpu.sync_copy(agg_s.at[pl.ds(r0, RPS)], aggp.at[c, pl.ds(r0, RPS)])

    @pl.when(s == NS - 1)
    def _():
        rt = NS * RPS
        pltpu.sync_copy(agg_s.at[pl.ds(rt, RTAIL)], aggp.at[c, pl.ds(rt, RTAIL)])


_sc_agg = pl.kernel(
    functools.partial(_sc_agg_body, True),
    out_type=(jax.ShapeDtypeStruct((NC, N, H), jnp.float32),
              jax.ShapeDtypeStruct((NW * N,), jnp.float32)),
    mesh=plsc.VectorSubcoreMesh(core_axis_name="c", subcore_axis_name="s"),
    scratch_types=[
        pltpu.VMEM((C,), jnp.int32),
        pltpu.VMEM((C,), jnp.int32),
        pltpu.VMEM((C,), jnp.int32),
        pltpu.VMEM((C,), jnp.int32),
        pltpu.VMEM((C,), jnp.int32),
        pltpu.VMEM((C,), jnp.int32),
        pltpu.VMEM((C, H), jnp.float32),
        pltpu.VMEM((C, H), jnp.float32),
        pltpu.VMEM((TAIL,), jnp.int32),
        pltpu.VMEM((TAIL,), jnp.int32),
        pltpu.VMEM((TAIL, H), jnp.float32),
        pltpu.VMEM((N,), jnp.float32),
        pltpu.VMEM_SHARED((N, H), jnp.float32),
        pltpu.SemaphoreType.DMA,
        pltpu.SemaphoreType.DMA,
        pltpu.SemaphoreType.DMA,
        pltpu.SemaphoreType.DMA,
        pltpu.SemaphoreType.DMA,
    ],
    compiler_params=pltpu.CompilerParams(needs_layout_passes=False),
)

_sc_agg_nd = pl.kernel(
    functools.partial(_sc_agg_body, False),
    out_type=(jax.ShapeDtypeStruct((NC, N, H), jnp.float32),),
    mesh=plsc.VectorSubcoreMesh(core_axis_name="c", subcore_axis_name="s"),
    scratch_types=[
        pltpu.VMEM((C,), jnp.int32),
        pltpu.VMEM((C,), jnp.int32),
        pltpu.VMEM((C,), jnp.int32),
        pltpu.VMEM((C,), jnp.int32),
        pltpu.VMEM((C,), jnp.int32),
        pltpu.VMEM((C,), jnp.int32),
        pltpu.VMEM((C, H), jnp.float32),
        pltpu.VMEM((C, H), jnp.float32),
        pltpu.VMEM((TAIL,), jnp.int32),
        pltpu.VMEM((TAIL,), jnp.int32),
        pltpu.VMEM((TAIL, H), jnp.float32),
        pltpu.VMEM_SHARED((N, H), jnp.float32),
        pltpu.SemaphoreType.DMA,
        pltpu.SemaphoreType.DMA,
        pltpu.SemaphoreType.DMA,
        pltpu.SemaphoreType.DMA,
        pltpu.SemaphoreType.DMA,
    ],
    compiler_params=pltpu.CompilerParams(needs_layout_passes=False),
)


def _sc_score_body(hitem, idx_all, bias,
                   rows_out, bvals_out,
                   idx_v, rows_v, bias_v, bch, gsem):
    c = lax.axis_index("c")
    s = lax.axis_index("s")
    wid = s * NC + c
    base = wid * QS
    pltpu.sync_copy(bias, bias_v)

    def chunk(k, carry):
        off = base + k * C
        pltpu.sync_copy(idx_all.at[pl.ds(off, C)], idx_v)
        pltpu.async_copy(hitem.at[idx_v], rows_v, gsem).wait()
        pltpu.sync_copy(rows_v, rows_out.at[pl.ds(off, C)])
        for j in range(C // L):
            bidx = idx_v[pl.ds(j * L, L)]
            bch[pl.ds(j * L, L)] = plsc.load_gather(bias_v, [bidx])
        pltpu.sync_copy(bch, bvals_out.at[pl.ds(off, C)])
        return carry

    lax.fori_loop(0, SCH, chunk, 0)


_sc_score = pl.kernel(
    _sc_score_body,
    out_type=(jax.ShapeDtypeStruct((4 * P, H), jnp.float32),
              jax.ShapeDtypeStruct((4 * P,), jnp.float32)),
    mesh=plsc.VectorSubcoreMesh(core_axis_name="c", subcore_axis_name="s"),
    scratch_types=[
        pltpu.VMEM((C,), jnp.int32),
        pltpu.VMEM((C, H), jnp.float32),
        pltpu.VMEM((N,), jnp.float32),
        pltpu.VMEM((C,), jnp.float32),
        pltpu.SemaphoreType.DMA,
    ],
    compiler_params=pltpu.CompilerParams(needs_layout_passes=False),
)


R = 2000
GRID = N // R


def _mm(a, b):
    return jnp.dot(a, b, preferred_element_type=jnp.float32)


def _tc_prep_body(x_ref, wp_ref, emb_ref, q1_ref, bq1_ref, h0_ref, n1_ref):
    h0 = _mm(x_ref[...], wp_ref[...]) + emb_ref[...]
    h0_ref[...] = h0
    n1_ref[...] = jnp.maximum(_mm(h0, q1_ref[...]) + bq1_ref[...], 0.0)


_tc_prep = pl.pallas_call(
    _tc_prep_body,
    grid=(GRID,),
    in_specs=[
        pl.BlockSpec((R, D), lambda i: (i, 0)),
        pl.BlockSpec((D, H), lambda i: (0, 0)),
        pl.BlockSpec((R, H), lambda i: (i, 0)),
        pl.BlockSpec((H, H), lambda i: (0, 0)),
        pl.BlockSpec((1, H), lambda i: (0, 0)),
    ],
    out_specs=[
        pl.BlockSpec((R, H), lambda i: (i, 0)),
        pl.BlockSpec((R, H), lambda i: (i, 0)),
    ],
    out_shape=[
        jax.ShapeDtypeStruct((N, H), jnp.float32),
        jax.ShapeDtypeStruct((N, H), jnp.float32),
    ],
)


def _tc_deg_body(degp_ref, deg_ref):
    deg_col = lax.dot_general(
        degp_ref[...], jnp.ones((NW, 1), jnp.float32),
        dimension_numbers=(((0,), (0,)), ((), ())),
        preferred_element_type=jnp.float32)
    deg_ref[...] = jnp.maximum(deg_col, 1.0)


_tc_deg = pl.pallas_call(
    _tc_deg_body,
    out_shape=jax.ShapeDtypeStruct((N, 1), jnp.float32),
)


def _combine(aggp_ref, deg, h_ref, w_ref, bw_ref):
    agg = aggp_ref[0] + aggp_ref[1]
    aggm = agg / deg
    z = jnp.maximum(
        _mm(aggm, w_ref[0:H]) + _mm(h_ref[...], w_ref[H:2 * H]) + bw_ref[...],
        0.0)
    zn = jnp.maximum(jnp.sqrt(jnp.sum(z * z, axis=1, keepdims=True)), 1e-6)
    return z / zn


def _tc_comb1_body(aggp_ref, degp_ref, h0_ref, w1_ref, bw1_ref, q2_ref,
                   bq2_ref, h1_ref, n2_ref):
    h1 = _combine(aggp_ref, degp_ref[...], h0_ref, w1_ref, bw1_ref)
    h1_ref[...] = h1
    n2_ref[...] = jnp.maximum(_mm(h1, q2_ref[...]) + bq2_ref[...], 0.0)


_tc_comb1 = pl.pallas_call(
    _tc_comb1_body,
    grid=(GRID,),
    in_specs=[
        pl.BlockSpec((NC, R, H), lambda i: (0, i, 0)),
        pl.BlockSpec((R, 1), lambda i: (i, 0)),
        pl.BlockSpec((R, H), lambda i: (i, 0)),
        pl.BlockSpec((2 * H, H), lambda i: (0, 0)),
        pl.BlockSpec((1, H), lambda i: (0, 0)),
        pl.BlockSpec((H, H), lambda i: (0, 0)),
        pl.BlockSpec((1, H), lambda i: (0, 0)),
    ],
    out_specs=[
        pl.BlockSpec((R, H), lambda i: (i, 0)),
        pl.BlockSpec((R, H), lambda i: (i, 0)),
    ],
    out_shape=[
        jax.ShapeDtypeStruct((N, H), jnp.float32),
        jax.ShapeDtypeStruct((N, H), jnp.float32),
    ],
)


def _tc_comb2_body(aggp_ref, degp_ref, h1_ref, h0_ref, w2_ref, bw2_ref,
                   hitem_ref):
    h2 = _combine(aggp_ref, degp_ref[...], h1_ref, w2_ref, bw2_ref)
    hitem_ref[...] = h0_ref[...] + h2


_tc_comb2 = pl.pallas_call(
    _tc_comb2_body,
    grid=(GRID,),
    in_specs=[
        pl.BlockSpec((NC, R, H), lambda i: (0, i, 0)),
        pl.BlockSpec((R, 1), lambda i: (i, 0)),
        pl.BlockSpec((R, H), lambda i: (i, 0)),
        pl.BlockSpec((R, H), lambda i: (i, 0)),
        pl.BlockSpec((2 * H, H), lambda i: (0, 0)),
        pl.BlockSpec((1, H), lambda i: (0, 0)),
    ],
    out_specs=pl.BlockSpec((R, H), lambda i: (i, 0)),
    out_shape=jax.ShapeDtypeStruct((N, H), jnp.float32),
)


RP = 2048
SGRID = P // RP


def _tc_score_body(rows_ref, bv_ref, out_ref):
    pos = jnp.sum(rows_ref[0] * rows_ref[1], axis=1) + bv_ref[0] + bv_ref[1]
    neg = jnp.sum(rows_ref[2] * rows_ref[3], axis=1) + bv_ref[2] + bv_ref[3]
    out_ref[...] = jnp.maximum(neg - pos + 1.0, 0.0)[None, :]


_tc_score = pl.pallas_call(
    _tc_score_body,
    grid=(SGRID,),
    in_specs=[
        pl.BlockSpec((4, RP, H), lambda i: (0, i, 0)),
        pl.BlockSpec((4, RP), lambda i: (0, i)),
    ],
    out_specs=pl.BlockSpec((1, RP), lambda i: (0, i)),
    out_shape=jax.ShapeDtypeStruct((1, P), jnp.float32),
)


def kernel(x, edge_index, pos_edges, neg_edges, item_emb, W_proj, Q1, bq1,
           W1, bw1, Q2, bq2, W2, bw2, bias):
    src = edge_index[0]
    dst = edge_index[1]
    zagg = jnp.zeros((N, H), jnp.float32)
    zdeg1 = jnp.zeros((N,), jnp.float32)

    h0, n1 = _tc_prep(x, W_proj, item_emb, Q1, bq1.reshape(1, H))
    aggp1, degf = _sc_agg(n1, src, dst, zagg, zdeg1)
    degp = _tc_deg(degf.reshape(NW, N))
    h1, n2 = _tc_comb1(aggp1, degp, h0, W1, bw1.reshape(1, H), Q2,
                       bq2.reshape(1, H))
    aggp2, = _sc_agg_nd(n2, src, dst, zagg)
    h_item = _tc_comb2(aggp2, degp, h1, h0, W2, bw2.reshape(1, H))

    idx_all = jnp.concatenate(
        [pos_edges[0], pos_edges[1], neg_edges[0], neg_edges[1]])
    rows, bvals = _sc_score(h_item, idx_all, bias)
    out = _tc_score(rows.reshape(4, P, H), bvals.reshape(4, P))
    return out.reshape(P)


# speedup vs baseline: 1.4992x; 1.0517x over previous
import functools

import jax
import jax.numpy as jnp
from jax import lax
from jax.experimental import pallas as pl
from jax.experimental.pallas import tpu as pltpu
from jax.experimental.pallas import tpu_sc as plsc

N = 10000
E = 320000
D = 128
H = 128
P = 8192

NC = 2
NS = 16
L = 16
NW = NC * NS
EPW = E // NW
C = 128
CA = 96
NCHUNK = EPW // CA
TAIL = EPW - NCHUNK * CA
RPS = 624
RTAIL = N - NS * RPS
DW = 16
QS = (4 * P) // NW
SCH = QS // C


def _sc_agg_body(with_deg, *refs):
    if with_deg:
        (table, src, dst, zagg, zdeg1, aggp, degp,
         sidx0, sidx1, sidx2, sidx3, didx0, didx1, didx2, didx3,
         rows0, rows1, rows2,
         tsidx, tdidx, trows, deg_v, agg_s,
         gsem0, gsem1, gsem2, ssem0, ssem1, ssem2,
         isem0, isem1, isem2, isem3) = refs
    else:
        (table, src, dst, zagg, aggp,
         sidx0, sidx1, sidx2, sidx3, didx0, didx1, didx2, didx3,
         rows0, rows1, rows2,
         tsidx, tdidx, trows, agg_s,
         gsem0, gsem1, gsem2, ssem0, ssem1, ssem2,
         isem0, isem1, isem2, isem3) = refs
    sidx = (sidx0, sidx1, sidx2, sidx3)
    didx = (didx0, didx1, didx2, didx3)
    rows = (rows0, rows1, rows2)
    gsem = (gsem0, gsem1, gsem2)
    ssem = (ssem0, ssem1, ssem2)
    isem = (isem0, isem1, isem2, isem3)
    c = lax.axis_index("c")
    s = lax.axis_index("s")
    wid = s * NC + c
    r0 = s * RPS
    pltpu.sync_copy(zagg.at[pl.ds(r0, RPS)], agg_s.at[pl.ds(r0, RPS)])
    if with_deg:
        pltpu.sync_copy(zdeg1, deg_v)

    @pl.when(s == NS - 1)
    def _():
        rt = NS * RPS
        pltpu.sync_copy(zagg.at[pl.ds(rt, RTAIL)], agg_s.at[pl.ds(rt, RTAIL)])

    plsc.subcore_barrier()

    base = wid * EPW
    ones16 = jnp.ones((L,), jnp.float32)

    def load_idx_async(k, qi):
        off = base + k * CA
        pltpu.async_copy(src.at[pl.ds(off, CA)], sidx[qi], isem[qi])
        pltpu.async_copy(dst.at[pl.ds(off, CA)], didx[qi], isem[qi])

    def wait_idx(qi):
        pltpu.make_async_copy(src.at[pl.ds(0, CA)], sidx[qi], isem[qi]).wait()
        pltpu.make_async_copy(dst.at[pl.ds(0, CA)], didx[qi], isem[qi]).wait()

    def wait_scat(rs, qs):
        pltpu.make_async_copy(rows[rs], agg_s.at[didx[qs]], ssem[rs]).wait()

    def step(k, rs, qs, g1=True, g3=True, g2=True):
        pltpu.make_async_copy(table.at[sidx[qs]], rows[rs], gsem[rs]).wait()
        pltpu.async_copy(rows[rs], agg_s.at[didx[qs]], ssem[rs], add=True)
        if with_deg:
            for j in range(CA // L):
                plsc.addupdate_scatter(deg_v, [didx[qs][pl.ds(j * L, L)]],
                                       ones16)
        if g1:
            wait_scat((rs + 2) % 3, (qs + 3) % 4)
        if g3:
            load_idx_async(k + 3, (qs + 3) % 4)
        if g2:
            wait_idx((qs + 2) % 4)
            pltpu.async_copy(table.at[sidx[(qs + 2) % 4]],
                             rows[(rs + 2) % 3], gsem[(rs + 2) % 3])

    load_idx_async(0, 0)
    load_idx_async(1, 1)
    load_idx_async(2, 2)
    wait_idx(0)
    pltpu.async_copy(table.at[sidx[0]], rows[0], gsem[0])
    wait_idx(1)
    pltpu.async_copy(table.at[sidx[1]], rows[1], gsem[1])

    for k in range(4):
        step(k, k % 3, k % 4, g1=(k >= 1))

    def outer(i, carry):
        k = 4 + 12 * i
        for u in range(12):
            step(k + u, (4 + u) % 3, u % 4)
        return carry

    lax.fori_loop(0, (NCHUNK - 8) // 12, outer, 0)

    for k in range(NCHUNK - 4, NCHUNK):
        step(k, k % 3, k % 4, g3=(k + 3 < NCHUNK), g2=(k + 2 < NCHUNK))
    wait_scat((NCHUNK - 1) % 3, (NCHUNK - 1) % 4)

    offt = base + NCHUNK * CA
    pltpu.sync_copy(src.at[pl.ds(offt, TAIL)], tsidx)
    pltpu.sync_copy(dst.at[pl.ds(offt, TAIL)], tdidx)
    pltpu.async_copy(table.at[tsidx], trows, gsem[0]).wait()
    pltpu.sync_copy(trows, agg_s.at[tdidx], add=True)
    if with_deg:
        plsc.addupdate_scatter(deg_v, [tdidx[...]], ones16)
        pltpu.sync_copy(deg_v, degp.at[pl.ds(wid * N, N)])

    plsc.subcore_barrier()
    pltpu.sync_copy(agg_s.at[pl.ds(r0, RPS)], aggp.at[c, pl.ds(r0, RPS)])

    @pl.when(s == NS - 1)
    def _():
        rt = NS * RPS
        pltpu.sync_copy(agg_s.at[pl.ds(rt, RTAIL)], aggp.at[c, pl.ds(rt, RTAIL)])


_IDX4 = [pltpu.VMEM((CA,), jnp.int32)] * 8
_ROWS4 = [pltpu.VMEM((CA, H), jnp.float32)] * 3
_TAILB = [pltpu.VMEM((TAIL,), jnp.int32),
          pltpu.VMEM((TAIL,), jnp.int32),
          pltpu.VMEM((TAIL, H), jnp.float32)]
_SEMS12 = [pltpu.SemaphoreType.DMA] * 10

_sc_agg = pl.kernel(
    functools.partial(_sc_agg_body, True),
    out_type=(jax.ShapeDtypeStruct((NC, N, H), jnp.float32),
              jax.ShapeDtypeStruct((NW * N,), jnp.float32)),
    mesh=plsc.VectorSubcoreMesh(core_axis_name="c", subcore_axis_name="s"),
    scratch_types=(_IDX4 + _ROWS4 + _TAILB
                   + [pltpu.VMEM((N,), jnp.float32),
                      pltpu.VMEM_SHARED((N, H), jnp.float32)]
                   + _SEMS12),
    compiler_params=pltpu.CompilerParams(needs_layout_passes=False),
)

_sc_agg_nd = pl.kernel(
    functools.partial(_sc_agg_body, False),
    out_type=(jax.ShapeDtypeStruct((NC, N, H), jnp.float32),),
    mesh=plsc.VectorSubcoreMesh(core_axis_name="c", subcore_axis_name="s"),
    scratch_types=(_IDX4 + _ROWS4 + _TAILB
                   + [pltpu.VMEM_SHARED((N, H), jnp.float32)]
                   + _SEMS12),
    compiler_params=pltpu.CompilerParams(needs_layout_passes=False),
)


def _sc_score_body(hitem, idx_all, bias,
                   rows_out, bvals_out,
                   idx_v, rows_v, bias_v, bch, gsem):
    c = lax.axis_index("c")
    s = lax.axis_index("s")
    wid = s * NC + c
    base = wid * QS
    pltpu.sync_copy(bias, bias_v)

    def chunk(k, carry):
        off = base + k * C
        pltpu.sync_copy(idx_all.at[pl.ds(off, C)], idx_v)
        pltpu.async_copy(hitem.at[idx_v], rows_v, gsem).wait()
        pltpu.sync_copy(rows_v, rows_out.at[pl.ds(off, C)])
        for j in range(C // L):
            bidx = idx_v[pl.ds(j * L, L)]
            bch[pl.ds(j * L, L)] = plsc.load_gather(bias_v, [bidx])
        pltpu.sync_copy(bch, bvals_out.at[pl.ds(off, C)])
        return carry

    lax.fori_loop(0, SCH, chunk, 0)


_sc_score = pl.kernel(
    _sc_score_body,
    out_type=(jax.ShapeDtypeStruct((4 * P, H), jnp.float32),
              jax.ShapeDtypeStruct((4 * P,), jnp.float32)),
    mesh=plsc.VectorSubcoreMesh(core_axis_name="c", subcore_axis_name="s"),
    scratch_types=[
        pltpu.VMEM((C,), jnp.int32),
        pltpu.VMEM((C, H), jnp.float32),
        pltpu.VMEM((N,), jnp.float32),
        pltpu.VMEM((C,), jnp.float32),
        pltpu.SemaphoreType.DMA,
    ],
    compiler_params=pltpu.CompilerParams(needs_layout_passes=False),
)


R = 2000
GRID = N // R


def _mm(a, b):
    return jnp.dot(a, b, preferred_element_type=jnp.float32)


def _tc_prep_body(x_ref, wp_ref, emb_ref, q1_ref, bq1_ref, h0_ref, n1_ref):
    h0 = _mm(x_ref[...], wp_ref[...]) + emb_ref[...]
    h0_ref[...] = h0
    n1_ref[...] = jnp.maximum(_mm(h0, q1_ref[...]) + bq1_ref[...], 0.0)


_tc_prep = pl.pallas_call(
    _tc_prep_body,
    grid=(GRID,),
    in_specs=[
        pl.BlockSpec((R, D), lambda i: (i, 0)),
        pl.BlockSpec((D, H), lambda i: (0, 0)),
        pl.BlockSpec((R, H), lambda i: (i, 0)),
        pl.BlockSpec((H, H), lambda i: (0, 0)),
        pl.BlockSpec((1, H), lambda i: (0, 0)),
    ],
    out_specs=[
        pl.BlockSpec((R, H), lambda i: (i, 0)),
        pl.BlockSpec((R, H), lambda i: (i, 0)),
    ],
    out_shape=[
        jax.ShapeDtypeStruct((N, H), jnp.float32),
        jax.ShapeDtypeStruct((N, H), jnp.float32),
    ],
)


def _tc_deg_body(degp_ref, deg_ref):
    deg_col = lax.dot_general(
        degp_ref[...], jnp.ones((NW, 1), jnp.float32),
        dimension_numbers=(((0,), (0,)), ((), ())),
        preferred_element_type=jnp.float32)
    deg_ref[...] = jnp.maximum(deg_col, 1.0)


_tc_deg = pl.pallas_call(
    _tc_deg_body,
    out_shape=jax.ShapeDtypeStruct((N, 1), jnp.float32),
)


def _combine(aggp_ref, deg, h_ref, w_ref, bw_ref):
    agg = aggp_ref[0] + aggp_ref[1]
    aggm = agg / deg
    z = jnp.maximum(
        _mm(aggm, w_ref[0:H]) + _mm(h_ref[...], w_ref[H:2 * H]) + bw_ref[...],
        0.0)
    zn = jnp.maximum(jnp.sqrt(jnp.sum(z * z, axis=1, keepdims=True)), 1e-6)
    return z / zn


def _tc_comb1_body(aggp_ref, degp_ref, h0_ref, w1_ref, bw1_ref, q2_ref,
                   bq2_ref, h1_ref, n2_ref):
    h1 = _combine(aggp_ref, degp_ref[...], h0_ref, w1_ref, bw1_ref)
    h1_ref[...] = h1
    n2_ref[...] = jnp.maximum(_mm(h1, q2_ref[...]) + bq2_ref[...], 0.0)


_tc_comb1 = pl.pallas_call(
    _tc_comb1_body,
    grid=(GRID,),
    in_specs=[
        pl.BlockSpec((NC, R, H), lambda i: (0, i, 0)),
        pl.BlockSpec((R, 1), lambda i: (i, 0)),
        pl.BlockSpec((R, H), lambda i: (i, 0)),
        pl.BlockSpec((2 * H, H), lambda i: (0, 0)),
        pl.BlockSpec((1, H), lambda i: (0, 0)),
        pl.BlockSpec((H, H), lambda i: (0, 0)),
        pl.BlockSpec((1, H), lambda i: (0, 0)),
    ],
    out_specs=[
        pl.BlockSpec((R, H), lambda i: (i, 0)),
        pl.BlockSpec((R, H), lambda i: (i, 0)),
    ],
    out_shape=[
        jax.ShapeDtypeStruct((N, H), jnp.float32),
        jax.ShapeDtypeStruct((N, H), jnp.float32),
    ],
)


def _tc_comb2_body(aggp_ref, degp_ref, h1_ref, h0_ref, w2_ref, bw2_ref,
                   hitem_ref):
    h2 = _combine(aggp_ref, degp_ref[...], h1_ref, w2_ref, bw2_ref)
    hitem_ref[...] = h0_ref[...] + h2


_tc_comb2 = pl.pallas_call(
    _tc_comb2_body,
    grid=(GRID,),
    in_specs=[
        pl.BlockSpec((NC, R, H), lambda i: (0, i, 0)),
        pl.BlockSpec((R, 1), lambda i: (i, 0)),
        pl.BlockSpec((R, H), lambda i: (i, 0)),
        pl.BlockSpec((R, H), lambda i: (i, 0)),
        pl.BlockSpec((2 * H, H), lambda i: (0, 0)),
        pl.BlockSpec((1, H), lambda i: (0, 0)),
    ],
    out_specs=pl.BlockSpec((R, H), lambda i: (i, 0)),
    out_shape=jax.ShapeDtypeStruct((N, H), jnp.float32),
)


RP = 2048
SGRID = P // RP


def _tc_score_body(rows_ref, bv_ref, out_ref):
    pos = jnp.sum(rows_ref[0] * rows_ref[1], axis=1) + bv_ref[0] + bv_ref[1]
    neg = jnp.sum(rows_ref[2] * rows_ref[3], axis=1) + bv_ref[2] + bv_ref[3]
    out_ref[...] = jnp.maximum(neg - pos + 1.0, 0.0)[None, :]


_tc_score = pl.pallas_call(
    _tc_score_body,
    grid=(SGRID,),
    in_specs=[
        pl.BlockSpec((4, RP, H), lambda i: (0, i, 0)),
        pl.BlockSpec((4, RP), lambda i: (0, i)),
    ],
    out_specs=pl.BlockSpec((1, RP), lambda i: (0, i)),
    out_shape=jax.ShapeDtypeStruct((1, P), jnp.float32),
)


def kernel(x, edge_index, pos_edges, neg_edges, item_emb, W_proj, Q1, bq1,
           W1, bw1, Q2, bq2, W2, bw2, bias):
    src = edge_index[0]
    dst = edge_index[1]
    zagg = jnp.zeros((N, H), jnp.float32)
    zdeg1 = jnp.zeros((N,), jnp.float32)

    h0, n1 = _tc_prep(x, W_proj, item_emb, Q1, bq1.reshape(1, H))
    aggp1, degf = _sc_agg(n1, src, dst, zagg, zdeg1)
    degp = _tc_deg(degf.reshape(NW, N))
    h1, n2 = _tc_comb1(aggp1, degp, h0, W1, bw1.reshape(1, H), Q2,
                       bq2.reshape(1, H))
    aggp2, = _sc_agg_nd(n2, src, dst, zagg)
    h_item = _tc_comb2(aggp2, degp, h1, h0, W2, bw2.reshape(1, H))

    idx_all = jnp.concatenate(
        [pos_edges[0], pos_edges[1], neg_edges[0], neg_edges[1]])
    rows, bvals = _sc_score(h_item, idx_all, bias)
    out = _tc_score(rows.reshape(4, P, H), bvals.reshape(4, P))
    return out.reshape(P)
